# baseline (speedup 1.0000x reference)
import functools

import jax
import jax.numpy as jnp
from jax import lax
from jax.experimental import pallas as pl
from jax.experimental.pallas import tpu as pltpu
from jax.experimental.pallas import tpu_sc as plsc

N_NODES = 10000
N_EDGES = 320000
D_FEAT = 128

NC = 2
NS = 16
NW = NC * NS

E_PER_W = N_EDGES // NW
CHUNK = 40
CHUNKS_PER_W = E_PER_W // CHUNK
N_CHUNKS = N_EDGES // CHUNK
NP = 10
NB = 5
N_BODIES = CHUNKS_PER_W // NP
SLAB = 640
LAST_SLAB = N_NODES - SLAB * (NS - 1)
ZROWS = 40
LOOK = 4
LAG = NB - LOOK


def _sc_partials(feat, src3, dst3):
    mesh = plsc.VectorSubcoreMesh(
        core_axis_name="c", subcore_axis_name="s", num_cores=NC, num_subcores=NS
    )

    scratch = (
        [pltpu.VMEM((1, CHUNK), jnp.int32) for _ in range(2 * NP)]
        + [pltpu.VMEM((CHUNK, D_FEAT), jnp.float32) for _ in range(NB)]
        + [pltpu.SemaphoreType.DMA for _ in range(NP + 2 * NB)]
        + [pltpu.VMEM_SHARED((N_NODES, D_FEAT), jnp.float32)]
    )

    @functools.partial(
        pl.kernel,
        out_type=jax.ShapeDtypeStruct((NC, N_NODES, D_FEAT), jnp.float32),
        mesh=mesh,
        scratch_types=scratch,
    )
    def k(feat_hbm, src_hbm, dst_hbm, part_hbm, *sc):
        srcb = sc[0:NP]
        dstb = sc[NP:2 * NP]
        rows = sc[2 * NP:2 * NP + NB]
        si = sc[2 * NP + NB:2 * NP + NB + NP]
        sg = sc[2 * NP + NB + NP:2 * NP + NB + NP + NB]
        ss = sc[2 * NP + NB + NP + NB:2 * NP + NB + NP + 2 * NB]
        acc = sc[-1]

        c = lax.axis_index("c")
        s = lax.axis_index("s")
        w = s * NC + c
        base = w * CHUNKS_PER_W

        zvec = jnp.zeros((16,), jnp.float32)

        def zrow(r, carry):
            for k16 in range(D_FEAT // 16):
                rows[0][r, pl.ds(k16 * 16, 16)] = zvec
            return carry

        lax.fori_loop(0, ZROWS, zrow, 0)

        @pl.when(s < NS - 1)
        def _():
            for i in range(SLAB // ZROWS):
                pltpu.sync_copy(
                    rows[0].at[pl.ds(0, ZROWS)],
                    acc.at[pl.ds(s * SLAB + i * ZROWS, ZROWS)],
                )

        @pl.when(s == NS - 1)
        def _():
            for i in range(LAST_SLAB // ZROWS):
                pltpu.sync_copy(
                    rows[0].at[pl.ds(0, ZROWS)],
                    acc.at[pl.ds((NS - 1) * SLAB + i * ZROWS, ZROWS)],
                )

        def load_idx(chunk, p):
            pltpu.async_copy(src_hbm.at[chunk], srcb[p], si[p])
            pltpu.async_copy(dst_hbm.at[chunk], dstb[p], si[p])

        def wait_idx(p):
            pltpu.make_async_copy(src_hbm.at[0], srcb[p], si[p]).wait()
            pltpu.make_async_copy(dst_hbm.at[0], dstb[p], si[p]).wait()

        def wait_gather(b):
            pltpu.make_async_copy(
                feat_hbm.at[pl.ds(0, CHUNK)], rows[b], sg[b]
            ).wait()

        def drain_scatter(b):
            pltpu.make_async_copy(
                feat_hbm.at[pl.ds(0, CHUNK)], rows[b], ss[b]
            ).wait()

        for p in range(NP):
            load_idx(base + p, p)
        for p in range(LOOK):
            wait_idx(p)
            pltpu.async_copy(feat_hbm.at[srcb[p].at[0]], rows[p], sg[p])

        plsc.subcore_barrier()

        def emit_body(jbase, first, last):
            for p in range(NP):
                b = p % NB
                wait_gather(b)
                pltpu.async_copy(
                    rows[b], acc.at[dstb[p].at[0]], ss[b], add=True
                )
                if not (first and p < LAG):
                    drain_scatter((p + LOOK) % NB)
                    if not (last and p >= LAG):
                        load_idx(jbase + p + (NP - LAG), (p + NP - LAG) % NP)
                if not (last and p >= NP - LOOK):
                    q2 = (p + LOOK) % NP
                    wait_idx(q2)
                    pltpu.async_copy(
                        feat_hbm.at[srcb[q2].at[0]],
                        rows[(p + LOOK) % NB],
                        sg[(p + LOOK) % NB],
                    )

        emit_body(base, first=True, last=False)

        def mid(t, carry):
            emit_body(base + t * NP, first=False, last=False)
            return carry

        lax.fori_loop(1, N_BODIES - 1, mid, 0)
        emit_body(base + (N_BODIES - 1) * NP, first=False, last=True)

        for q in range(LAG):
            drain_scatter((CHUNKS_PER_W - LAG + q) % NB)
        plsc.subcore_barrier()

        @pl.when(s < NS - 1)
        def _():
            pltpu.sync_copy(
                acc.at[pl.ds(s * SLAB, SLAB)],
                part_hbm.at[c, pl.ds(s * SLAB, SLAB)],
            )

        @pl.when(s == NS - 1)
        def _():
            pltpu.sync_copy(
                acc.at[pl.ds((NS - 1) * SLAB, LAST_SLAB)],
                part_hbm.at[c, pl.ds((NS - 1) * SLAB, LAST_SLAB)],
            )

    return k(feat, src3, dst3)


def _tc_combine_body(eps_ref, feat_ref, part_ref, o_ref):
    o_ref[...] = (1.0 + eps_ref[0]) * feat_ref[...] + part_ref[0] + part_ref[1]


def _tc_combine(eps, feat, part):
    rows = 1000
    grid = N_NODES // rows
    return pl.pallas_call(
        _tc_combine_body,
        grid=(grid,),
        in_specs=[
            pl.BlockSpec(memory_space=pltpu.SMEM),
            pl.BlockSpec((rows, D_FEAT), lambda i: (i, 0)),
            pl.BlockSpec((NC, rows, D_FEAT), lambda i: (0, i, 0)),
        ],
        out_specs=pl.BlockSpec((rows, D_FEAT), lambda i: (i, 0)),
        out_shape=jax.ShapeDtypeStruct((N_NODES, D_FEAT), jnp.float32),
    )(eps, feat, part)


@jax.jit
def kernel(feat, edge_index, eps):
    src3 = edge_index[0].astype(jnp.int32).reshape(N_CHUNKS, 1, CHUNK)
    dst3 = edge_index[1].astype(jnp.int32).reshape(N_CHUNKS, 1, CHUNK)
    part = _sc_partials(feat, src3, dst3)
    return _tc_combine(eps, feat, part)


# speedup vs baseline: 1.0869x; 1.0869x over previous
import functools

import jax
import jax.numpy as jnp
from jax import lax
from jax.experimental import pallas as pl
from jax.experimental.pallas import tpu as pltpu
from jax.experimental.pallas import tpu_sc as plsc

N_NODES = 10000
N_EDGES = 320000
D_FEAT = 128

NC = 2
NS = 16
NW = NC * NS

E_PER_W = N_EDGES // NW
CHUNK = 40
CHUNKS_PER_W = E_PER_W // CHUNK
N_CHUNKS = N_EDGES // CHUNK
NP = 10
NB = 5
N_BODIES = CHUNKS_PER_W // NP
SLAB = 640
LAST_SLAB = N_NODES - SLAB * (NS - 1)
ZROWS = 40
LOOK = 4
LAG = NB - LOOK


def _sc_partials(feat, e4):
    mesh = plsc.VectorSubcoreMesh(
        core_axis_name="c", subcore_axis_name="s", num_cores=NC, num_subcores=NS
    )

    scratch = (
        [pltpu.VMEM((1, CHUNK), jnp.int32) for _ in range(2 * NP)]
        + [pltpu.VMEM((CHUNK, D_FEAT), jnp.float32) for _ in range(NB)]
        + [pltpu.SemaphoreType.DMA for _ in range(NP + 2 * NB)]
        + [pltpu.VMEM_SHARED((N_NODES, D_FEAT), jnp.float32)]
    )

    @functools.partial(
        pl.kernel,
        out_type=jax.ShapeDtypeStruct((NC, N_NODES, D_FEAT), jnp.float32),
        mesh=mesh,
        scratch_types=scratch,
    )
    def k(feat_hbm, e_hbm, part_hbm, *sc):
        srcb = sc[0:NP]
        dstb = sc[NP:2 * NP]
        rows = sc[2 * NP:2 * NP + NB]
        si = sc[2 * NP + NB:2 * NP + NB + NP]
        sg = sc[2 * NP + NB + NP:2 * NP + NB + NP + NB]
        ss = sc[2 * NP + NB + NP + NB:2 * NP + NB + NP + 2 * NB]
        acc = sc[-1]

        c = lax.axis_index("c")
        s = lax.axis_index("s")
        w = s * NC + c
        base = w * CHUNKS_PER_W

        zvec = jnp.zeros((16,), jnp.float32)

        def zrow(r, carry):
            for k16 in range(D_FEAT // 16):
                rows[0][r, pl.ds(k16 * 16, 16)] = zvec
            return carry

        lax.fori_loop(0, ZROWS, zrow, 0)

        @pl.when(s < NS - 1)
        def _():
            for i in range(SLAB // ZROWS):
                pltpu.async_copy(
                    rows[0].at[pl.ds(0, ZROWS)],
                    acc.at[pl.ds(s * SLAB + i * ZROWS, ZROWS)],
                    ss[i % NB],
                )
            for i in range(SLAB // ZROWS):
                pltpu.make_async_copy(
                    feat_hbm.at[pl.ds(0, ZROWS)],
                    rows[0].at[pl.ds(0, ZROWS)],
                    ss[i % NB],
                ).wait()

        @pl.when(s == NS - 1)
        def _():
            for i in range(LAST_SLAB // ZROWS):
                pltpu.async_copy(
                    rows[0].at[pl.ds(0, ZROWS)],
                    acc.at[pl.ds((NS - 1) * SLAB + i * ZROWS, ZROWS)],
                    ss[i % NB],
                )
            for i in range(LAST_SLAB // ZROWS):
                pltpu.make_async_copy(
                    feat_hbm.at[pl.ds(0, ZROWS)],
                    rows[0].at[pl.ds(0, ZROWS)],
                    ss[i % NB],
                ).wait()

        def load_idx(chunk, p):
            pltpu.async_copy(e_hbm.at[0, chunk], srcb[p], si[p])
            pltpu.async_copy(e_hbm.at[1, chunk], dstb[p], si[p])

        def wait_idx(p):
            pltpu.make_async_copy(e_hbm.at[0, 0], srcb[p], si[p]).wait()
            pltpu.make_async_copy(e_hbm.at[1, 0], dstb[p], si[p]).wait()

        def wait_gather(b):
            pltpu.make_async_copy(
                feat_hbm.at[pl.ds(0, CHUNK)], rows[b], sg[b]
            ).wait()

        def drain_scatter(b):
            pltpu.make_async_copy(
                feat_hbm.at[pl.ds(0, CHUNK)], rows[b], ss[b]
            ).wait()

        for p in range(NP):
            load_idx(base + p, p)
        for p in range(LOOK):
            wait_idx(p)
            pltpu.async_copy(feat_hbm.at[srcb[p].at[0]], rows[p], sg[p])

        plsc.subcore_barrier()

        def emit_body(jbase, first, last):
            for p in range(NP):
                b = p % NB
                wait_gather(b)
                pltpu.async_copy(
                    rows[b], acc.at[dstb[p].at[0]], ss[b], add=True
                )
                if not (first and p < LAG):
                    drain_scatter((p + LOOK) % NB)
                    if not (last and p >= LAG):
                        load_idx(jbase + p + (NP - LAG), (p + NP - LAG) % NP)
                if not (last and p >= NP - LOOK):
                    q2 = (p + LOOK) % NP
                    wait_idx(q2)
                    pltpu.async_copy(
                        feat_hbm.at[srcb[q2].at[0]],
                        rows[(p + LOOK) % NB],
                        sg[(p + LOOK) % NB],
                    )

        emit_body(base, first=True, last=False)

        def mid(t, carry):
            emit_body(base + t * NP, first=False, last=False)
            return carry

        lax.fori_loop(1, N_BODIES - 1, mid, 0)
        emit_body(base + (N_BODIES - 1) * NP, first=False, last=True)

        for q in range(LAG):
            drain_scatter((CHUNKS_PER_W - LAG + q) % NB)
        plsc.subcore_barrier()

        @pl.when(s < NS - 1)
        def _():
            pltpu.sync_copy(
                acc.at[pl.ds(s * SLAB, SLAB)],
                part_hbm.at[c, pl.ds(s * SLAB, SLAB)],
            )

        @pl.when(s == NS - 1)
        def _():
            pltpu.sync_copy(
                acc.at[pl.ds((NS - 1) * SLAB, LAST_SLAB)],
                part_hbm.at[c, pl.ds((NS - 1) * SLAB, LAST_SLAB)],
            )

    return k(feat, e4)


def _tc_combine_body(eps_ref, feat_ref, part_ref, o_ref):
    o_ref[...] = (1.0 + eps_ref[0]) * feat_ref[...] + part_ref[0] + part_ref[1]


def _tc_combine(eps, feat, part):
    rows = 1000
    grid = N_NODES // rows
    return pl.pallas_call(
        _tc_combine_body,
        grid=(grid,),
        in_specs=[
            pl.BlockSpec(memory_space=pltpu.SMEM),
            pl.BlockSpec((rows, D_FEAT), lambda i: (i, 0)),
            pl.BlockSpec((NC, rows, D_FEAT), lambda i: (0, i, 0)),
        ],
        out_specs=pl.BlockSpec((rows, D_FEAT), lambda i: (i, 0)),
        out_shape=jax.ShapeDtypeStruct((N_NODES, D_FEAT), jnp.float32),
    )(eps, feat, part)


@jax.jit
def kernel(feat, edge_index, eps):
    e4 = edge_index.astype(jnp.int32).reshape(2, N_CHUNKS, 1, CHUNK)
    part = _sc_partials(feat, e4)
    return _tc_combine(eps, feat, part)
